# baseline (speedup 1.0000x reference)
import jax
import jax.numpy as jnp
from jax import lax
from jax.experimental import pallas as pl
from jax.experimental.pallas import tpu as pltpu
from jax.experimental.pallas import tpu_sc as plsc

N = 10000
E = 320000
HID = 128
NPAD = 10240
NC = 2
NS = 16
L = 16
EP1 = E // NS
EP2 = E // (NC * NS)
CH1 = 2000
NCH1 = EP1 // CH1
CH2 = 16
NCH2 = EP2 // CH2
NB = NPAD // NS
EPS = 1e-16


def _sc_exp(x):
    y = jnp.maximum(x, -87.0) * 1.4426950408889634
    k = (y + 12582912.0) - 12582912.0
    r = (y - k) * 0.6931471805599453
    p = 1.0 + r * (1.0 + r * (0.5 + r * (
        0.16666666666666666 + r * (0.041666666666666664 + r * (
            0.008333333333333333 + r * (0.001388888888888889
                                        + r * 0.0001984126984126984))))))
    ki = k.astype(jnp.int32) + 127
    scale = plsc.bitcast(ki << 23, jnp.float32)
    return p * scale


def _zero_1d(ref, n):
    z = jnp.zeros((L,), jnp.float32)

    def body(i, _):
        ref[pl.ds(i * L, L)] = z
        return 0

    lax.fori_loop(0, n // L, body, 0)


def _cs_scatter_add(acc_v, idxv, val, psb, ksb):
    k, v = plsc.sort_key_val(idxv, val)
    ps = plsc.cumsum(v)
    cnt, last = plsc.scan_count(k)
    psb[...] = ps
    ksb[...] = k
    j0 = jnp.arange(L, dtype=jnp.int32) - cnt
    kj0 = plsc.load_gather(ksb, [jnp.maximum(j0, 0)])
    j = jnp.where(kj0 == k, j0 - 1, j0)
    prev = plsc.load_gather(psb, [jnp.maximum(j, 0)])
    prev = jnp.where(j < 0, 0.0, prev)
    plsc.addupdate_scatter(acc_v, [k], ps - prev, mask=last)


def _reduce_to(s, part_v, dsh, racc, rtmp):
    pltpu.sync_copy(part_v, dsh.at[s])
    plsc.subcore_barrier()
    _zero_1d(racc, NB)
    for i in range(NS):
        pltpu.sync_copy(dsh.at[i, pl.ds(s * NB, NB)], rtmp)

        def addv(k, _):
            racc[pl.ds(k * L, L)] = racc[pl.ds(k * L, L)] + rtmp[pl.ds(k * L, L)]
            return 0

        lax.fori_loop(0, NB // L, addv, 0)


def _alpha_phase(s, src_hbm, dst_hbm, t_hbm, s_v, den_v, srcb, dstb, tb, vb,
                 psb, ksb, gsh, dsh, dm, racc, rtmp):
    p1b = s * EP1

    def chunk_a(ch, m):
        off = p1b + ch * CH1
        pltpu.sync_copy(src_hbm.at[pl.ds(off, CH1)], srcb)
        pltpu.sync_copy(dst_hbm.at[pl.ds(off, CH1)], dstb)
        pltpu.sync_copy(t_hbm.at[pl.ds(off, CH1)], tb)

        def vec_a(j, m):
            sv = plsc.load_gather(s_v, [srcb[pl.ds(j * L, L)]])
            dv = plsc.load_gather(s_v, [dstb[pl.ds(j * L, L)]])
            alpha = sv + dv + tb[pl.ds(j * L, L)]
            return jnp.maximum(m, alpha)

        return lax.fori_loop(0, CH1 // L, vec_a, m)

    m = lax.fori_loop(0, NCH1, chunk_a, jnp.full((L,), -1e30, jnp.float32))

    vb[...] = m
    pltpu.sync_copy(vb, gsh.at[s])
    plsc.subcore_barrier()
    gm = jnp.full((L,), -1e30, jnp.float32)
    for i in range(NS):
        pltpu.sync_copy(gsh.at[i], vb)
        gm = jnp.maximum(gm, vb[...])
    gmax = jnp.max(gm)

    _zero_1d(den_v, NPAD)

    def chunk_b(ch, _):
        off = p1b + ch * CH1
        pltpu.sync_copy(src_hbm.at[pl.ds(off, CH1)], srcb)
        pltpu.sync_copy(dst_hbm.at[pl.ds(off, CH1)], dstb)
        pltpu.sync_copy(t_hbm.at[pl.ds(off, CH1)], tb)

        def vec_b(j, _):
            sv = plsc.load_gather(s_v, [srcb[pl.ds(j * L, L)]])
            dstv = dstb[pl.ds(j * L, L)]
            dv = plsc.load_gather(s_v, [dstv])
            alpha = sv + dv + tb[pl.ds(j * L, L)]
            e = _sc_exp(alpha - gmax)
            _cs_scatter_add(den_v, dstv, e, psb, ksb)
            return 0

        lax.fori_loop(0, CH1 // L, vec_b, 0)
        return 0

    lax.fori_loop(0, NCH1, chunk_b, 0)

    _reduce_to(s, den_v, dsh, racc, rtmp)
    pltpu.sync_copy(racc, dm.at[pl.ds(s * NB, NB)])
    plsc.subcore_barrier()
    pltpu.sync_copy(dm, den_v)
    return gmax


def _sc_layer1(src_hbm, dst_hbm, t_hbm, s_hbm, x0_hbm, x1_hbm, acc_hbm,
               s_v, den_v, srcb, dstb, tb, vb, psb, ksb, racc, rtmp,
               x0_v, x1_v, a0_v, a1_v, ac_v, gsh, dsh, dm):
    c = lax.axis_index("c")
    s = lax.axis_index("s")

    pltpu.sync_copy(s_hbm, s_v)
    pltpu.sync_copy(x0_hbm, x0_v)
    pltpu.sync_copy(x1_hbm, x1_v)

    gmax = _alpha_phase(s, src_hbm, dst_hbm, t_hbm, s_v, den_v, srcb, dstb,
                        tb, vb, psb, ksb, gsh, dsh, dm, racc, rtmp)

    _zero_1d(a0_v, NPAD)
    _zero_1d(a1_v, NPAD)
    _zero_1d(ac_v, NPAD)
    p2b = s * EP1 + c * EP2

    def chunk_c(ch, _):
        off = p2b + ch * CH1
        pltpu.sync_copy(src_hbm.at[pl.ds(off, CH1)], srcb)
        pltpu.sync_copy(dst_hbm.at[pl.ds(off, CH1)], dstb)
        pltpu.sync_copy(t_hbm.at[pl.ds(off, CH1)], tb)

        def vec_c(j, _):
            srcv = srcb[pl.ds(j * L, L)]
            dstv = dstb[pl.ds(j * L, L)]
            sv = plsc.load_gather(s_v, [srcv])
            dv = plsc.load_gather(s_v, [dstv])
            alpha = sv + dv + tb[pl.ds(j * L, L)]
            e = _sc_exp(alpha - gmax)
            den = plsc.load_gather(den_v, [dstv])
            w = e / (den + EPS)
            g0 = plsc.load_gather(x0_v, [srcv])
            g1 = plsc.load_gather(x1_v, [srcv])
            _cs_scatter_add(a0_v, dstv, w * g0, psb, ksb)
            _cs_scatter_add(a1_v, dstv, w * g1, psb, ksb)
            _cs_scatter_add(ac_v, dstv, w, psb, ksb)
            return 0

        lax.fori_loop(0, CH1 // L, vec_c, 0)
        return 0

    lax.fori_loop(0, EP2 // CH1, chunk_c, 0)

    for k, arr in enumerate((a0_v, a1_v, ac_v)):
        plsc.subcore_barrier()
        _reduce_to(s, arr, dsh, racc, rtmp)
        pltpu.sync_copy(
            racc, acc_hbm.at[pl.ds((c * 3 + k) * NPAD + s * NB, NB)])


def _sc_layer2(src_hbm, dst_hbm, t_hbm, s_hbm, hw_hbm, out_hbm,
               s_v, den_v, srcb, dstb, tb, vb, psb, ksb, racc, rtmp,
               idxs, idxd, tb2, rowbuf, zbuf, gsh, dsh, dm, outacc, sem):
    c = lax.axis_index("c")
    s = lax.axis_index("s")

    pltpu.sync_copy(s_hbm, s_v)

    zv = jnp.zeros((L,), jnp.float32)
    for i in range(L):
        for r in range(HID // L):
            zbuf[i, pl.ds(r * L, L)] = zv

    def zrow(r, _):
        pltpu.sync_copy(zbuf, outacc.at[pl.ds(s * NB + r * L, L), :])
        return 0

    lax.fori_loop(0, NB // L, zrow, 0)

    gmax = _alpha_phase(s, src_hbm, dst_hbm, t_hbm, s_v, den_v, srcb, dstb,
                        tb, vb, psb, ksb, gsh, dsh, dm, racc, rtmp)

    p2b = s * EP1 + c * EP2

    def chunk_c(ch, _):
        off = p2b + ch * CH2
        pltpu.sync_copy(src_hbm.at[pl.ds(off, CH2)], idxs)
        pltpu.sync_copy(dst_hbm.at[pl.ds(off, CH2)], idxd)
        pltpu.sync_copy(t_hbm.at[pl.ds(off, CH2)], tb2)
        pltpu.async_copy(hw_hbm.at[idxs], rowbuf, sem).wait()
        srcv = idxs[...]
        dstv = idxd[...]
        sv = plsc.load_gather(s_v, [srcv])
        dv = plsc.load_gather(s_v, [dstv])
        alpha = sv + dv + tb2[...]
        e = _sc_exp(alpha - gmax)
        den = plsc.load_gather(den_v, [dstv])
        wv = e / (den + EPS)
        for k in range(L):
            av = jnp.full((L,), wv[k])
            for r in range(HID // L):
                rowbuf[k, pl.ds(r * L, L)] = rowbuf[k, pl.ds(r * L, L)] * av
        pltpu.sync_copy(rowbuf, outacc.at[idxd], add=True)
        return 0

    lax.fori_loop(0, NCH2, chunk_c, 0)
    plsc.subcore_barrier()

    def wrow(r, _):
        row = s * NB + r * L
        pltpu.sync_copy(outacc.at[pl.ds(row, L), :], zbuf)
        pltpu.sync_copy(zbuf, out_hbm.at[pl.ds(c * NPAD + row, L), :])
        return 0

    lax.fori_loop(0, NB // L, wrow, 0)


def _tc_node_pre(p_ref, x0_ref, x1_ref, s1_ref):
    s1_ref[...] = x0_ref[...] * p_ref[0] + x1_ref[...] * p_ref[1] + p_ref[2]


def _tc_edge_pre(p_ref, e0, e1, e2, e3, t1_ref, t2_ref):
    a0, a1, a2, a3 = e0[...], e1[...], e2[...], e3[...]
    pen = jnp.where(a1 == 1.0, 1e9, 0.0)
    t1_ref[...] = (a0 * p_ref[3] + a1 * p_ref[4] + a2 * p_ref[5]
                   + a3 * p_ref[6] + p_ref[7] - pen)
    t2_ref[...] = (a0 * p_ref[8] + a1 * p_ref[9] + a2 * p_ref[10]
                   + a3 * p_ref[11] + p_ref[12] - pen)


def _elu(x):
    return jnp.where(x > 0, x, jnp.exp(jnp.minimum(x, 0.0)) - 1.0)


def _tc_mid(acc0_ref, acc1_ref, waug_ref, w2_ref, b2_ref, a2p_ref,
            hw_ref, s2_ref):
    a = acc0_ref[...] + acc1_ref[...]
    out1 = jnp.dot(a, waug_ref[...], preferred_element_type=jnp.float32)
    h2 = _elu(out1)
    hw = jnp.dot(h2, w2_ref[...], preferred_element_type=jnp.float32) + b2_ref[...]
    hw_ref[...] = hw
    s2_ref[...] = jnp.dot(hw, a2p_ref[...], preferred_element_type=jnp.float32)


def _tc_post(p_ref, o0_ref, o1_ref, whp_ref, y_ref):
    h3 = _elu(o0_ref[...] + o1_ref[...])
    y_ref[...] = jnp.dot(h3, whp_ref[...],
                         preferred_element_type=jnp.float32) + p_ref[0]


def _pad_nodes(v):
    return jnp.pad(v, (0, NPAD - N))


def kernel(x, edge_index, edge_attr, W1, b1, We1, be1, att1, W2, b2, We2, be2,
           att2, Wh, bh):
    f32 = jnp.float32
    src = edge_index[0]
    dst = edge_index[1]
    a1 = att1[0]
    a2 = att2[0]

    v1 = W1 @ a1
    c1 = jnp.dot(b1, a1)
    u1 = We1 @ a1
    d1 = jnp.dot(be1, a1)
    u2 = We2 @ a2
    d2 = jnp.dot(be2, a2)
    params = jnp.concatenate([
        v1, jnp.stack([c1]), u1, jnp.stack([d1]), u2, jnp.stack([d2]),
        jnp.zeros((3,), f32)]).astype(f32)

    x0 = _pad_nodes(x[:, 0])
    x1p = _pad_nodes(x[:, 1])
    x0_2d = x0.reshape(NPAD // HID, HID)
    x1_2d = x1p.reshape(NPAD // HID, HID)
    ea = [edge_attr[:, i].reshape(E // HID, HID) for i in range(4)]

    s1_2d = pl.pallas_call(
        _tc_node_pre,
        out_shape=jax.ShapeDtypeStruct((NPAD // HID, HID), f32),
        in_specs=[pl.BlockSpec(memory_space=pltpu.SMEM),
                  pl.BlockSpec((NPAD // HID, HID), lambda: (0, 0)),
                  pl.BlockSpec((NPAD // HID, HID), lambda: (0, 0))],
        out_specs=pl.BlockSpec((NPAD // HID, HID), lambda: (0, 0)),
    )(params, x0_2d, x1_2d)
    s1 = s1_2d.reshape(NPAD)

    eb = E // HID
    t1_2d, t2_2d = pl.pallas_call(
        _tc_edge_pre,
        out_shape=[jax.ShapeDtypeStruct((E // HID, HID), f32)] * 2,
        in_specs=[pl.BlockSpec(memory_space=pltpu.SMEM)] +
                 [pl.BlockSpec((eb, HID), lambda: (0, 0))] * 4,
        out_specs=[pl.BlockSpec((eb, HID), lambda: (0, 0))] * 2,
    )(params, *ea)
    t1 = t1_2d.reshape(E)
    t2 = t2_2d.reshape(E)

    mesh = plsc.VectorSubcoreMesh(core_axis_name="c", subcore_axis_name="s")
    sc_params = pltpu.CompilerParams(needs_layout_passes=False)

    sc1 = pl.kernel(
        _sc_layer1,
        out_type=jax.ShapeDtypeStruct((NC * 3 * NPAD,), f32),
        mesh=mesh,
        compiler_params=sc_params,
        scratch_types=[
            pltpu.VMEM((NPAD,), f32),
            pltpu.VMEM((NPAD,), f32),
            pltpu.VMEM((CH1,), jnp.int32),
            pltpu.VMEM((CH1,), jnp.int32),
            pltpu.VMEM((CH1,), f32),
            pltpu.VMEM((L,), f32),
            pltpu.VMEM((L,), f32),
            pltpu.VMEM((L,), jnp.int32),
            pltpu.VMEM((NB,), f32),
            pltpu.VMEM((NB,), f32),
            pltpu.VMEM((NPAD,), f32),
            pltpu.VMEM((NPAD,), f32),
            pltpu.VMEM((NPAD,), f32),
            pltpu.VMEM((NPAD,), f32),
            pltpu.VMEM((NPAD,), f32),
            pltpu.MemorySpace.VMEM_SHARED((NS, L), f32),
            pltpu.MemorySpace.VMEM_SHARED((NS, NPAD), f32),
            pltpu.MemorySpace.VMEM_SHARED((NPAD,), f32),
        ],
    )
    acc = sc1(src, dst, t1, s1, x0, x1p).reshape(NC, 3, NPAD)

    accm0 = jnp.zeros((NPAD, 8), f32).at[:, :3].set(acc[0].T)
    accm1 = jnp.zeros((NPAD, 8), f32).at[:, :3].set(acc[1].T)
    waug = jnp.zeros((8, HID), f32).at[0].set(W1[0]).at[1].set(W1[1]).at[2].set(b1)
    a2pad = jnp.zeros((HID, 8), f32).at[:, 0].set(a2)
    nblk = 512
    hw_2d, s2_2d = pl.pallas_call(
        _tc_mid,
        grid=(NPAD // nblk,),
        out_shape=[jax.ShapeDtypeStruct((NPAD, HID), f32),
                   jax.ShapeDtypeStruct((NPAD, 8), f32)],
        in_specs=[pl.BlockSpec((nblk, 8), lambda i: (i, 0)),
                  pl.BlockSpec((nblk, 8), lambda i: (i, 0)),
                  pl.BlockSpec((8, HID), lambda i: (0, 0)),
                  pl.BlockSpec((HID, HID), lambda i: (0, 0)),
                  pl.BlockSpec((1, HID), lambda i: (0, 0)),
                  pl.BlockSpec((HID, 8), lambda i: (0, 0))],
        out_specs=[pl.BlockSpec((nblk, HID), lambda i: (i, 0)),
                   pl.BlockSpec((nblk, 8), lambda i: (i, 0))],
    )(accm0, accm1, waug, W2, b2.reshape(1, HID), a2pad)
    s2 = s2_2d[:, 0]

    sc2 = pl.kernel(
        _sc_layer2,
        out_type=jax.ShapeDtypeStruct((NC * NPAD, HID), f32),
        mesh=mesh,
        compiler_params=sc_params,
        scratch_types=[
            pltpu.VMEM((NPAD,), f32),
            pltpu.VMEM((NPAD,), f32),
            pltpu.VMEM((CH1,), jnp.int32),
            pltpu.VMEM((CH1,), jnp.int32),
            pltpu.VMEM((CH1,), f32),
            pltpu.VMEM((L,), f32),
            pltpu.VMEM((L,), f32),
            pltpu.VMEM((L,), jnp.int32),
            pltpu.VMEM((NB,), f32),
            pltpu.VMEM((NB,), f32),
            pltpu.VMEM((CH2,), jnp.int32),
            pltpu.VMEM((CH2,), jnp.int32),
            pltpu.VMEM((CH2,), f32),
            pltpu.VMEM((CH2, HID), f32),
            pltpu.VMEM((L, HID), f32),
            pltpu.MemorySpace.VMEM_SHARED((NS, L), f32),
            pltpu.MemorySpace.VMEM_SHARED((NS, NPAD), f32),
            pltpu.MemorySpace.VMEM_SHARED((NPAD,), f32),
            pltpu.MemorySpace.VMEM_SHARED((NPAD, HID), f32),
            pltpu.SemaphoreType.DMA,
        ],
    )
    outp = sc2(src, dst, t2, s2, hw_2d).reshape(NC, NPAD, HID)

    params_post = jnp.stack([bh[0]]).astype(f32)
    whp = jnp.zeros((HID, 8), f32).at[:, 0].set(Wh[:, 0])
    y_2d = pl.pallas_call(
        _tc_post,
        grid=(NPAD // nblk,),
        out_shape=jax.ShapeDtypeStruct((NPAD, 8), f32),
        in_specs=[pl.BlockSpec(memory_space=pltpu.SMEM),
                  pl.BlockSpec((nblk, HID), lambda i: (i, 0)),
                  pl.BlockSpec((nblk, HID), lambda i: (i, 0)),
                  pl.BlockSpec((HID, 8), lambda i: (0, 0))],
        out_specs=pl.BlockSpec((nblk, 8), lambda i: (i, 0)),
    )(params_post, outp[0], outp[1], whp)

    return y_2d[:N, 0:1]


# speedup vs baseline: 2.1669x; 2.1669x over previous
import jax
import jax.numpy as jnp
from jax import lax
from jax.experimental import pallas as pl
from jax.experimental.pallas import tpu as pltpu
from jax.experimental.pallas import tpu_sc as plsc

N = 10000
E = 320000
HID = 128
NPAD = 10240
NC = 2
NS = 16
L = 16
EP1 = E // NS
EP2 = E // (NC * NS)
CH1 = 2000
NCH1 = EP1 // CH1
CH2 = 80
NCH2 = EP2 // CH2
NB = NPAD // NS
EPS = 1e-16


def _sc_exp(x):
    y = jnp.maximum(x, -87.0) * 1.4426950408889634
    k = (y + 12582912.0) - 12582912.0
    r = (y - k) * 0.6931471805599453
    p = 1.0 + r * (1.0 + r * (0.5 + r * (
        0.16666666666666666 + r * (0.041666666666666664 + r * (
            0.008333333333333333 + r * (0.001388888888888889
                                        + r * 0.0001984126984126984))))))
    ki = k.astype(jnp.int32) + 127
    scale = plsc.bitcast(ki << 23, jnp.float32)
    return p * scale


def _zero_1d(ref, n):
    z = jnp.zeros((L,), jnp.float32)

    def body(i, _):
        ref[pl.ds(i * L, L)] = z
        return 0

    lax.fori_loop(0, n // L, body, 0)


def _cs_scatter_add(acc_v, idxv, val, psb, ksb):
    k, v = plsc.sort_key_val(idxv, val)
    ps = plsc.cumsum(v)
    cnt, last = plsc.scan_count(k)
    psb[...] = ps
    ksb[...] = k
    j0 = jnp.arange(L, dtype=jnp.int32) - cnt
    kj0 = plsc.load_gather(ksb, [jnp.maximum(j0, 0)])
    j = jnp.where(kj0 == k, j0 - 1, j0)
    prev = plsc.load_gather(psb, [jnp.maximum(j, 0)])
    prev = jnp.where(j < 0, 0.0, prev)
    plsc.addupdate_scatter(acc_v, [k], ps - prev, mask=last)


def _reduce_to(s, part_v, dsh, racc, rtmp):
    H = NPAD // 2
    for h in range(2):
        pltpu.sync_copy(part_v.at[pl.ds(h * H, H)], dsh.at[s])
        plsc.subcore_barrier()

        @pl.when(s // (NS // 2) == h)
        def _():
            _zero_1d(racc, NB)
            base = s * NB - h * H
            for i in range(NS):
                pltpu.sync_copy(dsh.at[i, pl.ds(base, NB)], rtmp)

                def addv(k, _):
                    racc[pl.ds(k * L, L)] = (racc[pl.ds(k * L, L)]
                                             + rtmp[pl.ds(k * L, L)])
                    return 0

                lax.fori_loop(0, NB // L, addv, 0)

        plsc.subcore_barrier()


def _alpha_phase(s, src_hbm, dst_hbm, t_hbm, s_v, den_v, srcb, dstb, tb, vb,
                 psb, ksb, gsh, dsh, dm, racc, rtmp):
    p1b = s * EP1

    def chunk_a(ch, m):
        off = p1b + ch * CH1
        pltpu.sync_copy(src_hbm.at[pl.ds(off, CH1)], srcb)
        pltpu.sync_copy(dst_hbm.at[pl.ds(off, CH1)], dstb)
        pltpu.sync_copy(t_hbm.at[pl.ds(off, CH1)], tb)

        def vec_a(j, m):
            sv = plsc.load_gather(s_v, [srcb[pl.ds(j * L, L)]])
            dv = plsc.load_gather(s_v, [dstb[pl.ds(j * L, L)]])
            alpha = sv + dv + tb[pl.ds(j * L, L)]
            return jnp.maximum(m, alpha)

        return lax.fori_loop(0, CH1 // L, vec_a, m)

    m = lax.fori_loop(0, NCH1, chunk_a, jnp.full((L,), -1e30, jnp.float32))

    vb[...] = m
    pltpu.sync_copy(vb, gsh.at[s])
    plsc.subcore_barrier()
    gm = jnp.full((L,), -1e30, jnp.float32)
    for i in range(NS):
        pltpu.sync_copy(gsh.at[i], vb)
        gm = jnp.maximum(gm, vb[...])
    gmax = jnp.max(gm)

    _zero_1d(den_v, NPAD)

    def chunk_b(ch, _):
        off = p1b + ch * CH1
        pltpu.sync_copy(src_hbm.at[pl.ds(off, CH1)], srcb)
        pltpu.sync_copy(dst_hbm.at[pl.ds(off, CH1)], dstb)
        pltpu.sync_copy(t_hbm.at[pl.ds(off, CH1)], tb)

        def vec_b(j, _):
            sv = plsc.load_gather(s_v, [srcb[pl.ds(j * L, L)]])
            dstv = dstb[pl.ds(j * L, L)]
            dv = plsc.load_gather(s_v, [dstv])
            alpha = sv + dv + tb[pl.ds(j * L, L)]
            e = _sc_exp(alpha - gmax)
            _cs_scatter_add(den_v, dstv, e, psb, ksb)
            return 0

        lax.fori_loop(0, CH1 // L, vec_b, 0)
        return 0

    lax.fori_loop(0, NCH1, chunk_b, 0)

    _reduce_to(s, den_v, dsh, racc, rtmp)
    pltpu.sync_copy(racc, dm.at[pl.ds(s * NB, NB)])
    plsc.subcore_barrier()
    pltpu.sync_copy(dm, den_v)
    return gmax


def _sc_layer1(src_hbm, dst_hbm, t_hbm, s_hbm, x0_hbm, x1_hbm, acc_hbm,
               s_v, den_v, srcb, dstb, tb, vb, psb, ksb, racc, rtmp,
               x0_v, x1_v, a0_v, a1_v, ac_v, gsh, dsh, dm):
    c = lax.axis_index("c")
    s = lax.axis_index("s")

    pltpu.sync_copy(s_hbm, s_v)
    pltpu.sync_copy(x0_hbm, x0_v)
    pltpu.sync_copy(x1_hbm, x1_v)

    gmax = _alpha_phase(s, src_hbm, dst_hbm, t_hbm, s_v, den_v, srcb, dstb,
                        tb, vb, psb, ksb, gsh, dsh, dm, racc, rtmp)

    _zero_1d(a0_v, NPAD)
    _zero_1d(a1_v, NPAD)
    _zero_1d(ac_v, NPAD)
    p2b = s * EP1 + c * EP2

    def chunk_c(ch, _):
        off = p2b + ch * CH1
        pltpu.sync_copy(src_hbm.at[pl.ds(off, CH1)], srcb)
        pltpu.sync_copy(dst_hbm.at[pl.ds(off, CH1)], dstb)
        pltpu.sync_copy(t_hbm.at[pl.ds(off, CH1)], tb)

        def vec_c(j, _):
            srcv = srcb[pl.ds(j * L, L)]
            dstv = dstb[pl.ds(j * L, L)]
            sv = plsc.load_gather(s_v, [srcv])
            dv = plsc.load_gather(s_v, [dstv])
            alpha = sv + dv + tb[pl.ds(j * L, L)]
            e = _sc_exp(alpha - gmax)
            den = plsc.load_gather(den_v, [dstv])
            w = e / (den + EPS)
            g0 = plsc.load_gather(x0_v, [srcv])
            g1 = plsc.load_gather(x1_v, [srcv])
            _cs_scatter_add(a0_v, dstv, w * g0, psb, ksb)
            _cs_scatter_add(a1_v, dstv, w * g1, psb, ksb)
            _cs_scatter_add(ac_v, dstv, w, psb, ksb)
            return 0

        lax.fori_loop(0, CH1 // L, vec_c, 0)
        return 0

    lax.fori_loop(0, EP2 // CH1, chunk_c, 0)

    for k, arr in enumerate((a0_v, a1_v, ac_v)):
        plsc.subcore_barrier()
        _reduce_to(s, arr, dsh, racc, rtmp)
        pltpu.sync_copy(
            racc, acc_hbm.at[pl.ds((c * 3 + k) * NPAD + s * NB, NB)])


def _sc_layer2(src_hbm, dst_hbm, t_hbm, s_hbm, hw_hbm, out_hbm,
               s_v, den_v, srcb, dstb, tb, vb, psb, ksb, racc, rtmp,
               idxs, idxd, tb2, rowbuf, zbuf, gsh, dsh, dm, outacc, sem):
    c = lax.axis_index("c")
    s = lax.axis_index("s")

    pltpu.sync_copy(s_hbm, s_v)

    zv = jnp.zeros((L,), jnp.float32)
    for i in range(L):
        for r in range(HID // L):
            zbuf[i, pl.ds(r * L, L)] = zv

    def zrow(r, _):
        pltpu.sync_copy(zbuf, outacc.at[pl.ds(s * NB + r * L, L), :])
        return 0

    lax.fori_loop(0, NB // L, zrow, 0)

    gmax = _alpha_phase(s, src_hbm, dst_hbm, t_hbm, s_v, den_v, srcb, dstb,
                        tb, vb, psb, ksb, gsh, dsh, dm, racc, rtmp)

    p2b = s * EP1 + c * EP2

    def chunk_c(ch, _):
        off = p2b + ch * CH2
        pltpu.sync_copy(src_hbm.at[pl.ds(off, CH2)], idxs)
        dma = pltpu.async_copy(hw_hbm.at[idxs], rowbuf, sem)
        pltpu.sync_copy(dst_hbm.at[pl.ds(off, CH2)], idxd)
        pltpu.sync_copy(t_hbm.at[pl.ds(off, CH2)], tb2)
        ws = []
        for v in range(CH2 // L):
            srcv = idxs[pl.ds(v * L, L)]
            dstv = idxd[pl.ds(v * L, L)]
            sv = plsc.load_gather(s_v, [srcv])
            dv = plsc.load_gather(s_v, [dstv])
            alpha = sv + dv + tb2[pl.ds(v * L, L)]
            e = _sc_exp(alpha - gmax)
            den = plsc.load_gather(den_v, [dstv])
            ws.append(e / (den + EPS))
        dma.wait()
        for v in range(CH2 // L):
            wv = ws[v]
            for k in range(L):
                av = jnp.full((L,), wv[k])
                row = v * L + k
                for r in range(HID // L):
                    rowbuf[row, pl.ds(r * L, L)] = (
                        rowbuf[row, pl.ds(r * L, L)] * av)
        pltpu.sync_copy(rowbuf, outacc.at[idxd], add=True)
        return 0

    lax.fori_loop(0, NCH2, chunk_c, 0)
    plsc.subcore_barrier()

    def wrow(r, _):
        row = s * NB + r * L
        pltpu.sync_copy(outacc.at[pl.ds(row, L), :], zbuf)
        pltpu.sync_copy(zbuf, out_hbm.at[pl.ds(c * NPAD + row, L), :])
        return 0

    lax.fori_loop(0, NB // L, wrow, 0)


def _tc_node_pre(p_ref, x0_ref, x1_ref, s1_ref):
    s1_ref[...] = x0_ref[...] * p_ref[0] + x1_ref[...] * p_ref[1] + p_ref[2]


def _tc_edge_pre(p_ref, e0, e1, e2, e3, t1_ref, t2_ref):
    a0, a1, a2, a3 = e0[...], e1[...], e2[...], e3[...]
    pen = jnp.where(a1 == 1.0, 1e9, 0.0)
    t1_ref[...] = (a0 * p_ref[3] + a1 * p_ref[4] + a2 * p_ref[5]
                   + a3 * p_ref[6] + p_ref[7] - pen)
    t2_ref[...] = (a0 * p_ref[8] + a1 * p_ref[9] + a2 * p_ref[10]
                   + a3 * p_ref[11] + p_ref[12] - pen)


def _elu(x):
    return jnp.where(x > 0, x, jnp.exp(jnp.minimum(x, 0.0)) - 1.0)


def _tc_mid(acc0_ref, acc1_ref, waug_ref, w2_ref, b2_ref, a2p_ref,
            hw_ref, s2_ref):
    a = acc0_ref[...] + acc1_ref[...]
    out1 = jnp.dot(a, waug_ref[...], preferred_element_type=jnp.float32)
    h2 = _elu(out1)
    hw = jnp.dot(h2, w2_ref[...], preferred_element_type=jnp.float32) + b2_ref[...]
    hw_ref[...] = hw
    s2_ref[...] = jnp.dot(hw, a2p_ref[...], preferred_element_type=jnp.float32)


def _tc_post(p_ref, o0_ref, o1_ref, whp_ref, y_ref):
    h3 = _elu(o0_ref[...] + o1_ref[...])
    y_ref[...] = jnp.dot(h3, whp_ref[...],
                         preferred_element_type=jnp.float32) + p_ref[0]


def _pad_nodes(v):
    return jnp.pad(v, (0, NPAD - N))


def kernel(x, edge_index, edge_attr, W1, b1, We1, be1, att1, W2, b2, We2, be2,
           att2, Wh, bh):
    f32 = jnp.float32
    src = edge_index[0]
    dst = edge_index[1]
    a1 = att1[0]
    a2 = att2[0]

    v1 = W1 @ a1
    c1 = jnp.dot(b1, a1)
    u1 = We1 @ a1
    d1 = jnp.dot(be1, a1)
    u2 = We2 @ a2
    d2 = jnp.dot(be2, a2)
    params = jnp.concatenate([
        v1, jnp.stack([c1]), u1, jnp.stack([d1]), u2, jnp.stack([d2]),
        jnp.zeros((3,), f32)]).astype(f32)

    x0 = _pad_nodes(x[:, 0])
    x1p = _pad_nodes(x[:, 1])
    x0_2d = x0.reshape(NPAD // HID, HID)
    x1_2d = x1p.reshape(NPAD // HID, HID)
    ea = [edge_attr[:, i].reshape(E // HID, HID) for i in range(4)]

    s1_2d = pl.pallas_call(
        _tc_node_pre,
        out_shape=jax.ShapeDtypeStruct((NPAD // HID, HID), f32),
        in_specs=[pl.BlockSpec(memory_space=pltpu.SMEM),
                  pl.BlockSpec((NPAD // HID, HID), lambda: (0, 0)),
                  pl.BlockSpec((NPAD // HID, HID), lambda: (0, 0))],
        out_specs=pl.BlockSpec((NPAD // HID, HID), lambda: (0, 0)),
    )(params, x0_2d, x1_2d)
    s1 = s1_2d.reshape(NPAD)

    eb = E // HID
    t1_2d, t2_2d = pl.pallas_call(
        _tc_edge_pre,
        out_shape=[jax.ShapeDtypeStruct((E // HID, HID), f32)] * 2,
        in_specs=[pl.BlockSpec(memory_space=pltpu.SMEM)] +
                 [pl.BlockSpec((eb, HID), lambda: (0, 0))] * 4,
        out_specs=[pl.BlockSpec((eb, HID), lambda: (0, 0))] * 2,
    )(params, *ea)
    t1 = t1_2d.reshape(E)
    t2 = t2_2d.reshape(E)

    mesh = plsc.VectorSubcoreMesh(core_axis_name="c", subcore_axis_name="s")
    sc_params = pltpu.CompilerParams(needs_layout_passes=False)

    sc1 = pl.kernel(
        _sc_layer1,
        out_type=jax.ShapeDtypeStruct((NC * 3 * NPAD,), f32),
        mesh=mesh,
        compiler_params=sc_params,
        scratch_types=[
            pltpu.VMEM((NPAD,), f32),
            pltpu.VMEM((NPAD,), f32),
            pltpu.VMEM((CH1,), jnp.int32),
            pltpu.VMEM((CH1,), jnp.int32),
            pltpu.VMEM((CH1,), f32),
            pltpu.VMEM((L,), f32),
            pltpu.VMEM((L,), f32),
            pltpu.VMEM((L,), jnp.int32),
            pltpu.VMEM((NB,), f32),
            pltpu.VMEM((NB,), f32),
            pltpu.VMEM((NPAD,), f32),
            pltpu.VMEM((NPAD,), f32),
            pltpu.VMEM((NPAD,), f32),
            pltpu.VMEM((NPAD,), f32),
            pltpu.VMEM((NPAD,), f32),
            pltpu.MemorySpace.VMEM_SHARED((NS, L), f32),
            pltpu.MemorySpace.VMEM_SHARED((NS, NPAD // 2), f32),
            pltpu.MemorySpace.VMEM_SHARED((NPAD,), f32),
        ],
    )
    acc = sc1(src, dst, t1, s1, x0, x1p).reshape(NC, 3, NPAD)

    accm0 = jnp.zeros((NPAD, 8), f32).at[:, :3].set(acc[0].T)
    accm1 = jnp.zeros((NPAD, 8), f32).at[:, :3].set(acc[1].T)
    waug = jnp.zeros((8, HID), f32).at[0].set(W1[0]).at[1].set(W1[1]).at[2].set(b1)
    a2pad = jnp.zeros((HID, 8), f32).at[:, 0].set(a2)
    nblk = 512
    hw_2d, s2_2d = pl.pallas_call(
        _tc_mid,
        grid=(NPAD // nblk,),
        out_shape=[jax.ShapeDtypeStruct((NPAD, HID), f32),
                   jax.ShapeDtypeStruct((NPAD, 8), f32)],
        in_specs=[pl.BlockSpec((nblk, 8), lambda i: (i, 0)),
                  pl.BlockSpec((nblk, 8), lambda i: (i, 0)),
                  pl.BlockSpec((8, HID), lambda i: (0, 0)),
                  pl.BlockSpec((HID, HID), lambda i: (0, 0)),
                  pl.BlockSpec((1, HID), lambda i: (0, 0)),
                  pl.BlockSpec((HID, 8), lambda i: (0, 0))],
        out_specs=[pl.BlockSpec((nblk, HID), lambda i: (i, 0)),
                   pl.BlockSpec((nblk, 8), lambda i: (i, 0))],
    )(accm0, accm1, waug, W2, b2.reshape(1, HID), a2pad)
    s2 = s2_2d[:, 0]

    sc2 = pl.kernel(
        _sc_layer2,
        out_type=jax.ShapeDtypeStruct((NC * NPAD, HID), f32),
        mesh=mesh,
        compiler_params=sc_params,
        scratch_types=[
            pltpu.VMEM((NPAD,), f32),
            pltpu.VMEM((NPAD,), f32),
            pltpu.VMEM((CH1,), jnp.int32),
            pltpu.VMEM((CH1,), jnp.int32),
            pltpu.VMEM((CH1,), f32),
            pltpu.VMEM((L,), f32),
            pltpu.VMEM((L,), f32),
            pltpu.VMEM((L,), jnp.int32),
            pltpu.VMEM((NB,), f32),
            pltpu.VMEM((NB,), f32),
            pltpu.VMEM((CH2,), jnp.int32),
            pltpu.VMEM((CH2,), jnp.int32),
            pltpu.VMEM((CH2,), f32),
            pltpu.VMEM((CH2, HID), f32),
            pltpu.VMEM((L, HID), f32),
            pltpu.MemorySpace.VMEM_SHARED((NS, L), f32),
            pltpu.MemorySpace.VMEM_SHARED((NS, NPAD // 2), f32),
            pltpu.MemorySpace.VMEM_SHARED((NPAD,), f32),
            pltpu.MemorySpace.VMEM_SHARED((NPAD, HID), f32),
            pltpu.SemaphoreType.DMA,
        ],
    )
    outp = sc2(src, dst, t2, s2, hw_2d).reshape(NC, NPAD, HID)

    params_post = jnp.stack([bh[0]]).astype(f32)
    whp = jnp.zeros((HID, 8), f32).at[:, 0].set(Wh[:, 0])
    y_2d = pl.pallas_call(
        _tc_post,
        grid=(NPAD // nblk,),
        out_shape=jax.ShapeDtypeStruct((NPAD, 8), f32),
        in_specs=[pl.BlockSpec(memory_space=pltpu.SMEM),
                  pl.BlockSpec((nblk, HID), lambda i: (i, 0)),
                  pl.BlockSpec((nblk, HID), lambda i: (i, 0)),
                  pl.BlockSpec((HID, 8), lambda i: (0, 0))],
        out_specs=pl.BlockSpec((nblk, 8), lambda i: (i, 0)),
    )(params_post, outp[0], outp[1], whp)

    return y_2d[:N, 0:1]
